# baseline (speedup 1.0000x reference)
import functools

import numpy as np
import jax
import jax.numpy as jnp
from jax.experimental import pallas as pl
from jax.experimental.pallas import tpu as pltpu

_B_POS = 1
_B_OFF0 = 1


def _geo(S):
    Wp = max(8, -(-(S + 2) // 8) * 8)
    PF = (S + 4) * Wp
    A = 2 * Wp
    r_last = (S + 1) * Wp + S
    L = -(-(r_last + 1 - A) // 8) * 8
    assert A - Wp >= 0 and A + Wp + L <= PF
    return Wp, PF, A, L


def _vmask(S):
    Wp, PF, _, _ = _geo(S)
    idx = np.arange(PF)
    row, col = idx // Wp, idx % Wp
    m = (row >= 2) & (row < S + 2) & (col >= 1) & (col < S + 1)
    return jnp.asarray(m.astype(np.float32).reshape(PF, 1))


def _to_can(x, S, dtype):
    N, _, _, C = x.shape
    Wp, PF, _, _ = _geo(S)
    xp = jnp.pad(x, ((0, 0), (2, 2), (1, Wp - S - 1), (0, 0)))
    return xp.reshape(N, PF, C).astype(dtype)


def _patches7(x, S):
    N, _, _, Cin = x.shape
    xp = jnp.pad(x, ((0, 0), (3, 3), (3, 3), (0, 0)))
    cols = [xp[:, a:a + S, b:b + S, :] for a in range(7) for b in range(7)]
    pat = jnp.concatenate(cols, axis=-1)
    kkc = 49 * Cin
    kp = -(-kkc // 8) * 8
    if kp != kkc:
        pat = jnp.pad(pat, ((0, 0), (0, 0), (0, 0), (0, kp - kkc)))
    return _to_can(pat, S, jnp.bfloat16)


def _build_x3(load, x3_ref, F, C):
    x3_ref[:, C:2 * C] = load(0, F)
    x3_ref[0:8, 0:C] = jnp.zeros((8, C), x3_ref.dtype)
    x3_ref[8:F, 0:C] = load(7, F - 1)
    x3_ref[0:F - 8, 2 * C:3 * C] = load(1, F - 7)
    x3_ref[F - 8:F, 2 * C:3 * C] = jnp.zeros((8, C), x3_ref.dtype)


def _conv3(x3_ref, w_ref, b_ref, W0, Wp, Lc, cout):
    acc = jnp.broadcast_to(b_ref[...], (Lc, cout)).astype(jnp.float32)
    for t in range(3):
        s = W0 + (t - 1) * Wp
        acc = acc + jnp.dot(x3_ref[s:s + Lc, :], w_ref[t],
                            preferred_element_type=jnp.float32)
    return acc


def _pos_kernel(pat_ref, wh_ref, bh_ref, w1_ref, b1_ref, w2_ref, b2_ref,
                wt_ref, bt_ref, off_ref, xin_ref, m_ref, o_ref,
                pos_ref, x3_ref, h_ref, *, Wp, PF, B):
    F = pat_ref.shape[1]
    W0 = Wp
    Lc = F - 2 * Wp
    C = 128
    mask = m_ref[W0:W0 + Lc, :]

    acc = jnp.dot(pat_ref[0, W0:W0 + Lc, :], wh_ref[...],
                  preferred_element_type=jnp.float32)
    acc = (acc + bh_ref[...]) * mask
    pos_ref[0:W0, :] = jnp.zeros((W0, C), pos_ref.dtype)
    pos_ref[W0 + Lc:F, :] = jnp.zeros((F - W0 - Lc, C), pos_ref.dtype)
    pos_ref[W0:W0 + Lc, :] = acc.astype(pos_ref.dtype)
    h_ref[0:W0, :] = jnp.zeros((W0, C), h_ref.dtype)
    h_ref[W0 + Lc:F, :] = jnp.zeros((F - W0 - Lc, C), h_ref.dtype)

    for l in range(8):
        _build_x3(lambda lo, hi: pos_ref[lo:hi, :], x3_ref, F, C)
        a1 = _conv3(x3_ref, w1_ref[l], b1_ref[l], W0, Wp, Lc, C)
        h_ref[W0:W0 + Lc, :] = (jnp.maximum(a1, 0.0) * mask).astype(h_ref.dtype)
        _build_x3(lambda lo, hi: h_ref[lo:hi, :], x3_ref, F, C)
        a2 = _conv3(x3_ref, w2_ref[l], b2_ref[l], W0, Wp, Lc, C)
        a2 = a2 * mask + pos_ref[W0:W0 + Lc, :].astype(jnp.float32)
        pos_ref[W0:W0 + Lc, :] = a2.astype(pos_ref.dtype)

    _build_x3(lambda lo, hi: pos_ref[lo:hi, :], x3_ref, F, C)
    acc = _conv3(x3_ref, wt_ref, bt_ref, W0, Wp, Lc, 4)
    pos4 = jnp.tanh(acc)
    o_ref[0, 0:W0, :] = jnp.zeros((W0, 1), o_ref.dtype)
    o_ref[0, W0 + Lc:F, :] = jnp.zeros((F - W0 - Lc, 1), o_ref.dtype)
    for b in range(B):
        lo = max(W0, b * PF)
        hi = min(W0 + Lc, (b + 1) * PF)
        res = jnp.sum(pos4[lo - W0:hi - W0, :] * off_ref[0, b], axis=-1,
                      keepdims=True)
        o_ref[0, lo:hi, :] = xin_ref[0, lo:hi, :] + res


def _pos_call(pat, wh, bh, w1s, b1s, w2s, b2s, wt, bt, off, x_can, mask):
    G, F, KKC = pat.shape
    Wp, PF, _, _ = _geo(64)
    B = F // PF
    kern = functools.partial(_pos_kernel, Wp=Wp, PF=PF, B=B)
    return pl.pallas_call(
        kern,
        out_shape=jax.ShapeDtypeStruct((G, F, 1), jnp.float32),
        grid=(G,),
        in_specs=[
            pl.BlockSpec((1, F, KKC), lambda n: (n, 0, 0)),
            pl.BlockSpec((KKC, 128), lambda n: (0, 0)),
            pl.BlockSpec((1, 128), lambda n: (0, 0)),
            pl.BlockSpec((8, 3, 384, 128), lambda n: (0, 0, 0, 0)),
            pl.BlockSpec((8, 1, 128), lambda n: (0, 0, 0)),
            pl.BlockSpec((8, 3, 384, 128), lambda n: (0, 0, 0, 0)),
            pl.BlockSpec((8, 1, 128), lambda n: (0, 0, 0)),
            pl.BlockSpec((3, 384, 4), lambda n: (0, 0, 0)),
            pl.BlockSpec((1, 4), lambda n: (0, 0)),
            pl.BlockSpec((1, 8, 4), lambda n: (n, 0, 0)),
            pl.BlockSpec((1, F, 1), lambda n: (n, 0, 0)),
            pl.BlockSpec((F, 1), lambda n: (0, 0)),
        ],
        out_specs=pl.BlockSpec((1, F, 1), lambda n: (n, 0, 0)),
        scratch_shapes=[
            pltpu.VMEM((F, 128), jnp.bfloat16),
            pltpu.VMEM((F, 384), jnp.bfloat16),
            pltpu.VMEM((F, 128), jnp.bfloat16),
        ],
        compiler_params=pltpu.CompilerParams(dimension_semantics=("parallel",)),
    )(pat, wh, bh, w1s, b1s, w2s, b2s, wt, bt, off, x_can, mask)


def _off0_kernel(pat_ref, wh_ref, bh_ref, w1_ref, b1_ref, w2_ref, b2_ref,
                 m_ref, o_ref, act_ref, x3_ref, h_ref, o2_ref, m1_ref,
                 *, Wp):
    F = pat_ref.shape[1]
    W0 = Wp
    Lc = F - 2 * Wp
    C = 256
    mask = m_ref[W0:W0 + Lc, :]

    acc = jnp.dot(pat_ref[0, W0:W0 + Lc, :], wh_ref[...],
                  preferred_element_type=jnp.float32)
    acc = (acc + bh_ref[...]) * mask
    act_ref[0:W0, :] = jnp.zeros((W0, C), act_ref.dtype)
    act_ref[W0 + Lc:F, :] = jnp.zeros((F - W0 - Lc, C), act_ref.dtype)
    act_ref[W0:W0 + Lc, :] = acc.astype(act_ref.dtype)
    h_ref[0:W0, :] = jnp.zeros((W0, C), h_ref.dtype)
    h_ref[W0 + Lc:F, :] = jnp.zeros((F - W0 - Lc, C), h_ref.dtype)

    _build_x3(lambda lo, hi: act_ref[lo:hi, :], x3_ref, F, C)
    a1 = _conv3(x3_ref, w1_ref, b1_ref, W0, Wp, Lc, C)
    h_ref[W0:W0 + Lc, :] = (jnp.maximum(a1, 0.0) * mask).astype(h_ref.dtype)
    _build_x3(lambda lo, hi: h_ref[lo:hi, :], x3_ref, F, C)
    a2 = _conv3(x3_ref, w2_ref, b2_ref, W0, Wp, Lc, 128)
    o2_ref[0:W0, :] = jnp.zeros((W0, 128), o2_ref.dtype)
    o2_ref[W0 + Lc:F, :] = jnp.zeros((F - W0 - Lc, 128), o2_ref.dtype)
    o2_ref[W0:W0 + Lc, :] = (jnp.maximum(a2, 0.0) * mask).astype(o2_ref.dtype)

    m1_ref[0:W0, :] = jnp.zeros((W0, 128), m1_ref.dtype)
    m1_ref[W0 + Lc:F, :] = jnp.zeros((F - W0 - Lc, 128), m1_ref.dtype)
    m1_ref[W0:W0 + Lc, :] = jnp.maximum(o2_ref[W0:W0 + Lc, :],
                                        o2_ref[W0 + Wp:W0 + Wp + Lc, :])
    o_ref[0, W0:W0 + Lc, :] = jnp.maximum(m1_ref[W0:W0 + Lc, :],
                                          m1_ref[W0 + 1:W0 + 1 + Lc, :])
    o_ref[0, 0:W0, :] = jnp.zeros((W0, 128), o_ref.dtype)
    o_ref[0, W0 + Lc:F, :] = jnp.zeros((F - W0 - Lc, 128), o_ref.dtype)


def _off0_call(pat, wh, bh, w1, b1, w2, b2, mask):
    G, F, KKC = pat.shape
    Wp, _, _, _ = _geo(64)
    kern = functools.partial(_off0_kernel, Wp=Wp)
    return pl.pallas_call(
        kern,
        out_shape=jax.ShapeDtypeStruct((G, F, 128), jnp.bfloat16),
        grid=(G,),
        in_specs=[
            pl.BlockSpec((1, F, KKC), lambda n: (n, 0, 0)),
            pl.BlockSpec((KKC, 256), lambda n: (0, 0)),
            pl.BlockSpec((1, 256), lambda n: (0, 0)),
            pl.BlockSpec((3, 768, 256), lambda n: (0, 0, 0)),
            pl.BlockSpec((1, 256), lambda n: (0, 0)),
            pl.BlockSpec((3, 768, 128), lambda n: (0, 0, 0)),
            pl.BlockSpec((1, 128), lambda n: (0, 0)),
            pl.BlockSpec((F, 1), lambda n: (0, 0)),
        ],
        out_specs=pl.BlockSpec((1, F, 128), lambda n: (n, 0, 0)),
        scratch_shapes=[
            pltpu.VMEM((F, 256), jnp.bfloat16),
            pltpu.VMEM((F, 768), jnp.bfloat16),
            pltpu.VMEM((F, 256), jnp.bfloat16),
            pltpu.VMEM((F, 128), jnp.bfloat16),
            pltpu.VMEM((F, 128), jnp.bfloat16),
        ],
        compiler_params=pltpu.CompilerParams(dimension_semantics=("parallel",)),
    )(pat, wh, bh, w1, b1, w2, b2, mask)


def _stage_kernel(x_ref, w1_ref, b1_ref, w2_ref, b2_ref, m_ref, o_ref,
                  x3_ref, h_ref, o2_ref, m1_ref, *, Wp, W0, Lc, cin, cout):
    F = x_ref.shape[1]
    mask = m_ref[W0:W0 + Lc, :]

    _build_x3(lambda lo, hi: x_ref[0, lo:hi, :], x3_ref, F, cin)
    a1 = _conv3(x3_ref, w1_ref, b1_ref, W0, Wp, Lc, cin)
    h_ref[0:W0, :] = jnp.zeros((W0, cin), h_ref.dtype)
    h_ref[W0 + Lc:F, :] = jnp.zeros((F - W0 - Lc, cin), h_ref.dtype)
    h_ref[W0:W0 + Lc, :] = (jnp.maximum(a1, 0.0) * mask).astype(h_ref.dtype)
    _build_x3(lambda lo, hi: h_ref[lo:hi, :], x3_ref, F, cin)
    a2 = _conv3(x3_ref, w2_ref, b2_ref, W0, Wp, Lc, cout)
    o2_ref[0:W0, :] = jnp.zeros((W0, cout), o2_ref.dtype)
    o2_ref[W0 + Lc:F, :] = jnp.zeros((F - W0 - Lc, cout), o2_ref.dtype)
    o2_ref[W0:W0 + Lc, :] = (jnp.maximum(a2, 0.0) * mask).astype(o2_ref.dtype)

    m1_ref[0:W0, :] = jnp.zeros((W0, cout), m1_ref.dtype)
    m1_ref[W0 + Lc:F, :] = jnp.zeros((F - W0 - Lc, cout), m1_ref.dtype)
    m1_ref[W0:W0 + Lc, :] = jnp.maximum(o2_ref[W0:W0 + Lc, :],
                                        o2_ref[W0 + Wp:W0 + Wp + Lc, :])
    o_ref[0, W0:W0 + Lc, :] = jnp.maximum(m1_ref[W0:W0 + Lc, :],
                                          m1_ref[W0 + 1:W0 + 1 + Lc, :])
    o_ref[0, 0:W0, :] = jnp.zeros((W0, cout), o_ref.dtype)
    o_ref[0, W0 + Lc:F, :] = jnp.zeros((F - W0 - Lc, cout), o_ref.dtype)


def _stage_call(xs, w1, b1, w2, b2, maskS, S, B):
    G, F, cin = xs.shape
    cout = w2.shape[2]
    Wp, PF, _, _ = _geo(S)
    mask_st = jnp.tile(maskS, (B, 1))
    kern = functools.partial(_stage_kernel, Wp=Wp, W0=Wp, Lc=F - 2 * Wp,
                             cin=cin, cout=cout)
    return pl.pallas_call(
        kern,
        out_shape=jax.ShapeDtypeStruct((G, F, cout), jnp.bfloat16),
        grid=(G,),
        in_specs=[
            pl.BlockSpec((1, F, cin), lambda n: (n, 0, 0)),
            pl.BlockSpec((3, 3 * cin, cin), lambda n: (0, 0, 0)),
            pl.BlockSpec((1, cin), lambda n: (0, 0)),
            pl.BlockSpec((3, 3 * cin, cout), lambda n: (0, 0, 0)),
            pl.BlockSpec((1, cout), lambda n: (0, 0)),
            pl.BlockSpec((F, 1), lambda n: (0, 0)),
        ],
        out_specs=pl.BlockSpec((1, F, cout), lambda n: (n, 0, 0)),
        scratch_shapes=[
            pltpu.VMEM((F, 3 * cin), jnp.bfloat16),
            pltpu.VMEM((F, cin), jnp.bfloat16),
            pltpu.VMEM((F, cout), jnp.bfloat16),
            pltpu.VMEM((F, cout), jnp.bfloat16),
        ],
        compiler_params=pltpu.CompilerParams(dimension_semantics=("parallel",)),
    )(xs, w1, b1, w2, b2, mask_st)


def _downselect(allpos, S, B, B_next):
    G, F, C = allpos.shape
    N = G * B
    Wp, PF, _, _ = _geo(S)
    t = allpos.reshape(N, S + 4, Wp, C)[:, 2:S + 2:2, 1:S + 1:2, :]
    S2 = S // 2
    can = _to_can(t, S2, jnp.bfloat16)
    _, PF2, _ = can.shape
    return can.reshape(N // B_next, B_next * PF2, C)


def kernel(x, y, pos_head_w, pos_head_b, body_w1, body_b1, body_w2, body_b2,
           pos_tail_w, pos_tail_b, offset_head_w, offset_head_b,
           ob0_w1, ob0_b1, ob0_w2, ob0_b2,
           ob1_w1, ob1_b1, ob1_w2, ob1_b2,
           ob2_w1, ob2_b1, ob2_w2, ob2_b2,
           ob3_w1, ob3_b1, ob3_w2, ob3_b2,
           ob4_w1, ob4_b1, ob4_w2, ob4_b2,
           ob5_w1, ob5_b1, ob5_w2, ob5_b2):
    N = x.shape[0]
    bf = jnp.bfloat16
    mask64 = _vmask(64)
    Wp, PF, _, _ = _geo(64)

    def w3(w):
        _, _, ci, co = w.shape
        return w.reshape(3, 3 * ci, co).astype(bf)

    def whead(w):
        co = w.shape[-1]
        kkc = w.shape[0] * w.shape[1] * w.shape[2]
        wf = w.reshape(kkc, co)
        kp = -(-kkc // 8) * 8
        if kp != kkc:
            wf = jnp.pad(wf, ((0, kp - kkc), (0, 0)))
        return wf.astype(bf)

    def stack(a, B):
        n, pf, c = a.shape
        return a.reshape(n // B, B * pf, c)

    B0 = min(_B_OFF0, N)
    xy = jnp.concatenate([x, y], axis=-1)
    pato = stack(_patches7(xy, 64), B0)
    allp = _off0_call(pato, whead(offset_head_w), offset_head_b.reshape(1, -1),
                      w3(ob0_w1), ob0_b1.reshape(1, -1),
                      w3(ob0_w2), ob0_b2.reshape(1, -1),
                      jnp.tile(mask64, (B0, 1)))

    stages = [
        (ob1_w1, ob1_b1, ob1_w2, ob1_b2, 32, min(4, N)),
        (ob2_w1, ob2_b1, ob2_w2, ob2_b2, 16, min(16, N)),
        (ob3_w1, ob3_b1, ob3_w2, ob3_b2, 8, min(32, N)),
        (ob4_w1, ob4_b1, ob4_w2, ob4_b2, 4, min(64, N)),
        (ob5_w1, ob5_b1, ob5_w2, ob5_b2, 2, min(128, N)),
    ]
    cur = _downselect(allp, 64, B0, stages[0][5])
    for i, (w1, b1, w2, b2, S, B) in enumerate(stages):
        allp = _stage_call(cur, w3(w1), b1.reshape(1, -1),
                           w3(w2), b2.reshape(1, -1), _vmask(S), S, B)
        if i + 1 < len(stages):
            cur = _downselect(allp, S, B, stages[i + 1][5])
    Wp2 = _geo(2)[0]
    off = allp.reshape(N, 6, Wp2, 4)[:, 2:3, 1:2, :].astype(jnp.float32)

    Bp = min(_B_POS, N)
    pat = stack(_patches7(x, 64), Bp)
    x_can = stack(_to_can(x, 64, jnp.float32), Bp)
    out1_can = _pos_call(
        pat, whead(pos_head_w), pos_head_b.reshape(1, -1),
        body_w1.reshape(8, 3, 384, 128).astype(bf), body_b1.reshape(8, 1, 128),
        body_w2.reshape(8, 3, 384, 128).astype(bf), body_b2.reshape(8, 1, 128),
        w3(pos_tail_w), pos_tail_b.reshape(1, -1),
        jnp.pad(off.reshape(N // Bp, Bp, 4), ((0, 0), (0, 8 - Bp), (0, 0))),
        x_can, jnp.tile(mask64, (Bp, 1)))
    out1 = out1_can.reshape(N, 68, Wp, 1)[:, 2:66, 1:65, :]
    return out1, off


# speedup vs baseline: 1.3933x; 1.3933x over previous
import functools

import numpy as np
import jax
import jax.numpy as jnp
from jax.experimental import pallas as pl
from jax.experimental.pallas import tpu as pltpu


def _geo(S):
    Wp = max(8, -(-(S + 2) // 8) * 8)
    PF = (S + 4) * Wp
    A = 2 * Wp
    r_last = (S + 1) * Wp + S
    L = -(-(r_last + 1 - A) // 8) * 8
    assert A - Wp >= 0 and A + Wp + L <= PF
    return Wp, PF, A, L


def _vmask(S):
    Wp, PF, _, _ = _geo(S)
    idx = np.arange(PF)
    row, col = idx // Wp, idx % Wp
    m = (row >= 2) & (row < S + 2) & (col >= 1) & (col < S + 1)
    return jnp.asarray(m.astype(np.float32).reshape(PF, 1))


def _to_can(x, S, dtype):
    N, _, _, C = x.shape
    Wp, PF, _, _ = _geo(S)
    xp = jnp.pad(x, ((0, 0), (2, 2), (1, Wp - S - 1), (0, 0)))
    return xp.reshape(N, PF, C).astype(dtype)


def _patches7(x, S):
    N, _, _, Cin = x.shape
    xp = jnp.pad(x, ((0, 0), (3, 3), (3, 3), (0, 0)))
    cols = [xp[:, a:a + S, b:b + S, :] for a in range(7) for b in range(7)]
    pat = jnp.concatenate(cols, axis=-1)
    kkc = 49 * Cin
    kp = -(-kkc // 8) * 8
    if kp != kkc:
        pat = jnp.pad(pat, ((0, 0), (0, 0), (0, 0), (0, kp - kkc)))
    return _to_can(pat, S, jnp.bfloat16)


def _build_x3(load, x3_ref, F, C):
    x3_ref[:, C:2 * C] = load(0, F)
    x3_ref[0:8, 0:C] = jnp.zeros((8, C), x3_ref.dtype)
    x3_ref[8:F, 0:C] = load(7, F - 1)
    x3_ref[0:F - 8, 2 * C:3 * C] = load(1, F - 7)
    x3_ref[F - 8:F, 2 * C:3 * C] = jnp.zeros((8, C), x3_ref.dtype)


def _conv3(x3_ref, w_ref, b_ref, W0, Wp, Lc, cout):
    acc = jnp.broadcast_to(b_ref[...], (Lc, cout)).astype(jnp.float32)
    for t in range(3):
        s = W0 + (t - 1) * Wp
        acc = acc + jnp.dot(x3_ref[s:s + Lc, :], w_ref[t],
                            preferred_element_type=jnp.float32)
    return acc


def _pos_kernel(pat_ref, wh_ref, bh_ref, w1_ref, b1_ref, w2_ref, b2_ref,
                wt_ref, bt_ref, off_ref, xin_ref, m_ref, o_ref,
                pos_ref, x3_ref, h_ref, *, Wp, A, L):
    PF = pat_ref.shape[1]
    C = 128
    mask = m_ref[A:A + L, :]

    acc = jnp.dot(pat_ref[0, A:A + L, :], wh_ref[...],
                  preferred_element_type=jnp.float32)
    acc = (acc + bh_ref[...]) * mask
    pos_ref[0:A, :] = jnp.zeros((A, C), pos_ref.dtype)
    pos_ref[A + L:PF, :] = jnp.zeros((PF - A - L, C), pos_ref.dtype)
    pos_ref[A:A + L, :] = acc.astype(pos_ref.dtype)
    h_ref[0:A, :] = jnp.zeros((A, C), h_ref.dtype)
    h_ref[A + L:PF, :] = jnp.zeros((PF - A - L, C), h_ref.dtype)

    for l in range(8):
        _build_x3(lambda lo, hi: pos_ref[lo:hi, :], x3_ref, PF, C)
        a1 = _conv3(x3_ref, w1_ref[l], b1_ref[l], A, Wp, L, C)
        h_ref[A:A + L, :] = (jnp.maximum(a1, 0.0) * mask).astype(h_ref.dtype)
        _build_x3(lambda lo, hi: h_ref[lo:hi, :], x3_ref, PF, C)
        a2 = _conv3(x3_ref, w2_ref[l], b2_ref[l], A, Wp, L, C)
        a2 = a2 * mask + pos_ref[A:A + L, :].astype(jnp.float32)
        pos_ref[A:A + L, :] = a2.astype(pos_ref.dtype)

    _build_x3(lambda lo, hi: pos_ref[lo:hi, :], x3_ref, PF, C)
    acc = _conv3(x3_ref, wt_ref, bt_ref, A, Wp, L, 4)
    pos4 = jnp.tanh(acc)
    res = jnp.sum(pos4 * off_ref[0], axis=-1, keepdims=True)
    o_ref[0, A:A + L, :] = xin_ref[0, A:A + L, :] + res
    o_ref[0, 0:A, :] = jnp.zeros((A, 1), o_ref.dtype)
    o_ref[0, A + L:PF, :] = jnp.zeros((PF - A - L, 1), o_ref.dtype)


def _pos_call(pat, wh, bh, w1s, b1s, w2s, b2s, wt, bt, off, x_can, mask):
    N, PF, KKC = pat.shape
    Wp, PF2, A, L = _geo(64)
    assert PF == PF2
    kern = functools.partial(_pos_kernel, Wp=Wp, A=A, L=L)
    return pl.pallas_call(
        kern,
        out_shape=jax.ShapeDtypeStruct((N, PF, 1), jnp.float32),
        grid=(N,),
        in_specs=[
            pl.BlockSpec((1, PF, KKC), lambda n: (n, 0, 0)),
            pl.BlockSpec((KKC, 128), lambda n: (0, 0)),
            pl.BlockSpec((1, 128), lambda n: (0, 0)),
            pl.BlockSpec((8, 3, 384, 128), lambda n: (0, 0, 0, 0)),
            pl.BlockSpec((8, 1, 128), lambda n: (0, 0, 0)),
            pl.BlockSpec((8, 3, 384, 128), lambda n: (0, 0, 0, 0)),
            pl.BlockSpec((8, 1, 128), lambda n: (0, 0, 0)),
            pl.BlockSpec((3, 384, 4), lambda n: (0, 0, 0)),
            pl.BlockSpec((1, 4), lambda n: (0, 0)),
            pl.BlockSpec((1, 1, 4), lambda n: (n, 0, 0)),
            pl.BlockSpec((1, PF, 1), lambda n: (n, 0, 0)),
            pl.BlockSpec((PF, 1), lambda n: (0, 0)),
        ],
        out_specs=pl.BlockSpec((1, PF, 1), lambda n: (n, 0, 0)),
        scratch_shapes=[
            pltpu.VMEM((PF, 128), jnp.bfloat16),
            pltpu.VMEM((PF, 384), jnp.bfloat16),
            pltpu.VMEM((PF, 128), jnp.bfloat16),
        ],
        compiler_params=pltpu.CompilerParams(dimension_semantics=("parallel",)),
    )(pat, wh, bh, w1s, b1s, w2s, b2s, wt, bt, off, x_can, mask)


def _off0_kernel(pat_ref, wh_ref, bh_ref, w1_ref, b1_ref, w2_ref, b2_ref,
                 m_ref, o_ref, act_ref, x3_ref, h_ref, o2_ref, m1_ref,
                 *, Wp, A, L):
    PF = pat_ref.shape[1]
    C = 256
    mask = m_ref[A:A + L, :]

    acc = jnp.dot(pat_ref[0, A:A + L, :], wh_ref[...],
                  preferred_element_type=jnp.float32)
    acc = (acc + bh_ref[...]) * mask
    act_ref[0:A, :] = jnp.zeros((A, C), act_ref.dtype)
    act_ref[A + L:PF, :] = jnp.zeros((PF - A - L, C), act_ref.dtype)
    act_ref[A:A + L, :] = acc.astype(act_ref.dtype)
    h_ref[0:A, :] = jnp.zeros((A, C), h_ref.dtype)
    h_ref[A + L:PF, :] = jnp.zeros((PF - A - L, C), h_ref.dtype)

    _build_x3(lambda lo, hi: act_ref[lo:hi, :], x3_ref, PF, C)
    a1 = _conv3(x3_ref, w1_ref, b1_ref, A, Wp, L, C)
    h_ref[A:A + L, :] = (jnp.maximum(a1, 0.0) * mask).astype(h_ref.dtype)
    _build_x3(lambda lo, hi: h_ref[lo:hi, :], x3_ref, PF, C)
    a2 = _conv3(x3_ref, w2_ref, b2_ref, A, Wp, L, 128)
    o2_ref[0:A, :] = jnp.zeros((A, 128), o2_ref.dtype)
    o2_ref[A + L:PF, :] = jnp.zeros((PF - A - L, 128), o2_ref.dtype)
    o2_ref[A:A + L, :] = (jnp.maximum(a2, 0.0) * mask).astype(o2_ref.dtype)

    m1_ref[0:A, :] = jnp.zeros((A, 128), m1_ref.dtype)
    m1_ref[A + L:PF, :] = jnp.zeros((PF - A - L, 128), m1_ref.dtype)
    m1_ref[A:A + L, :] = jnp.maximum(o2_ref[A:A + L, :],
                                     o2_ref[A + Wp:A + Wp + L, :])
    o_ref[0, A:A + L, :] = jnp.maximum(m1_ref[A:A + L, :],
                                       m1_ref[A + 1:A + 1 + L, :])
    o_ref[0, 0:A, :] = jnp.zeros((A, 128), o_ref.dtype)
    o_ref[0, A + L:PF, :] = jnp.zeros((PF - A - L, 128), o_ref.dtype)


def _off0_call(pat, wh, bh, w1, b1, w2, b2, mask):
    N, PF, KKC = pat.shape
    Wp, _, A, L = _geo(64)
    kern = functools.partial(_off0_kernel, Wp=Wp, A=A, L=L)
    return pl.pallas_call(
        kern,
        out_shape=jax.ShapeDtypeStruct((N, PF, 128), jnp.bfloat16),
        grid=(N,),
        in_specs=[
            pl.BlockSpec((1, PF, KKC), lambda n: (n, 0, 0)),
            pl.BlockSpec((KKC, 256), lambda n: (0, 0)),
            pl.BlockSpec((1, 256), lambda n: (0, 0)),
            pl.BlockSpec((3, 768, 256), lambda n: (0, 0, 0)),
            pl.BlockSpec((1, 256), lambda n: (0, 0)),
            pl.BlockSpec((3, 768, 128), lambda n: (0, 0, 0)),
            pl.BlockSpec((1, 128), lambda n: (0, 0)),
            pl.BlockSpec((PF, 1), lambda n: (0, 0)),
        ],
        out_specs=pl.BlockSpec((1, PF, 128), lambda n: (n, 0, 0)),
        scratch_shapes=[
            pltpu.VMEM((PF, 256), jnp.bfloat16),
            pltpu.VMEM((PF, 768), jnp.bfloat16),
            pltpu.VMEM((PF, 256), jnp.bfloat16),
            pltpu.VMEM((PF, 128), jnp.bfloat16),
            pltpu.VMEM((PF, 128), jnp.bfloat16),
        ],
        compiler_params=pltpu.CompilerParams(dimension_semantics=("parallel",)),
    )(pat, wh, bh, w1, b1, w2, b2, mask)


def _stage_kernel(x_ref, w1_ref, b1_ref, w2_ref, b2_ref, m_ref, o_ref,
                  x3_ref, h_ref, o2_ref, m1_ref, *, Wp, W0, Lc, cin, cout):
    F = x_ref.shape[1]
    mask = m_ref[W0:W0 + Lc, :]

    _build_x3(lambda lo, hi: x_ref[0, lo:hi, :], x3_ref, F, cin)
    a1 = _conv3(x3_ref, w1_ref, b1_ref, W0, Wp, Lc, cin)
    h_ref[0:W0, :] = jnp.zeros((W0, cin), h_ref.dtype)
    h_ref[W0 + Lc:F, :] = jnp.zeros((F - W0 - Lc, cin), h_ref.dtype)
    h_ref[W0:W0 + Lc, :] = (jnp.maximum(a1, 0.0) * mask).astype(h_ref.dtype)
    _build_x3(lambda lo, hi: h_ref[lo:hi, :], x3_ref, F, cin)
    a2 = _conv3(x3_ref, w2_ref, b2_ref, W0, Wp, Lc, cout)
    o2_ref[0:W0, :] = jnp.zeros((W0, cout), o2_ref.dtype)
    o2_ref[W0 + Lc:F, :] = jnp.zeros((F - W0 - Lc, cout), o2_ref.dtype)
    o2_ref[W0:W0 + Lc, :] = (jnp.maximum(a2, 0.0) * mask).astype(o2_ref.dtype)

    m1_ref[0:W0, :] = jnp.zeros((W0, cout), m1_ref.dtype)
    m1_ref[W0 + Lc:F, :] = jnp.zeros((F - W0 - Lc, cout), m1_ref.dtype)
    m1_ref[W0:W0 + Lc, :] = jnp.maximum(o2_ref[W0:W0 + Lc, :],
                                        o2_ref[W0 + Wp:W0 + Wp + Lc, :])
    o_ref[0, W0:W0 + Lc, :] = jnp.maximum(m1_ref[W0:W0 + Lc, :],
                                          m1_ref[W0 + 1:W0 + 1 + Lc, :])
    o_ref[0, 0:W0, :] = jnp.zeros((W0, cout), o_ref.dtype)
    o_ref[0, W0 + Lc:F, :] = jnp.zeros((F - W0 - Lc, cout), o_ref.dtype)


def _stage_call(xs, w1, b1, w2, b2, maskS, S, B):
    G, F, cin = xs.shape
    cout = w2.shape[2]
    Wp, PF, _, _ = _geo(S)
    mask_st = jnp.tile(maskS, (B, 1))
    kern = functools.partial(_stage_kernel, Wp=Wp, W0=Wp, Lc=F - 2 * Wp,
                             cin=cin, cout=cout)
    return pl.pallas_call(
        kern,
        out_shape=jax.ShapeDtypeStruct((G, F, cout), jnp.bfloat16),
        grid=(G,),
        in_specs=[
            pl.BlockSpec((1, F, cin), lambda n: (n, 0, 0)),
            pl.BlockSpec((3, 3 * cin, cin), lambda n: (0, 0, 0)),
            pl.BlockSpec((1, cin), lambda n: (0, 0)),
            pl.BlockSpec((3, 3 * cin, cout), lambda n: (0, 0, 0)),
            pl.BlockSpec((1, cout), lambda n: (0, 0)),
            pl.BlockSpec((F, 1), lambda n: (0, 0)),
        ],
        out_specs=pl.BlockSpec((1, F, cout), lambda n: (n, 0, 0)),
        scratch_shapes=[
            pltpu.VMEM((F, 3 * cin), jnp.bfloat16),
            pltpu.VMEM((F, cin), jnp.bfloat16),
            pltpu.VMEM((F, cout), jnp.bfloat16),
            pltpu.VMEM((F, cout), jnp.bfloat16),
        ],
        compiler_params=pltpu.CompilerParams(dimension_semantics=("parallel",)),
    )(xs, w1, b1, w2, b2, mask_st)


def _downselect(allpos, S, B, B_next):
    G, F, C = allpos.shape
    N = G * B
    Wp, PF, _, _ = _geo(S)
    t = allpos.reshape(N, S + 4, Wp, C)[:, 2:S + 2:2, 1:S + 1:2, :]
    S2 = S // 2
    can = _to_can(t, S2, jnp.bfloat16)
    _, PF2, _ = can.shape
    return can.reshape(N // B_next, B_next * PF2, C)


def kernel(x, y, pos_head_w, pos_head_b, body_w1, body_b1, body_w2, body_b2,
           pos_tail_w, pos_tail_b, offset_head_w, offset_head_b,
           ob0_w1, ob0_b1, ob0_w2, ob0_b2,
           ob1_w1, ob1_b1, ob1_w2, ob1_b2,
           ob2_w1, ob2_b1, ob2_w2, ob2_b2,
           ob3_w1, ob3_b1, ob3_w2, ob3_b2,
           ob4_w1, ob4_b1, ob4_w2, ob4_b2,
           ob5_w1, ob5_b1, ob5_w2, ob5_b2):
    N = x.shape[0]
    bf = jnp.bfloat16
    mask64 = _vmask(64)

    def w3(w):
        _, _, ci, co = w.shape
        return w.reshape(3, 3 * ci, co).astype(bf)

    def whead(w):
        co = w.shape[-1]
        kkc = w.shape[0] * w.shape[1] * w.shape[2]
        wf = w.reshape(kkc, co)
        kp = -(-kkc // 8) * 8
        if kp != kkc:
            wf = jnp.pad(wf, ((0, kp - kkc), (0, 0)))
        return wf.astype(bf)

    xy = jnp.concatenate([x, y], axis=-1)
    pato = _patches7(xy, 64)
    allp = _off0_call(pato, whead(offset_head_w), offset_head_b.reshape(1, -1),
                      w3(ob0_w1), ob0_b1.reshape(1, -1),
                      w3(ob0_w2), ob0_b2.reshape(1, -1), mask64)

    stages = [
        (ob1_w1, ob1_b1, ob1_w2, ob1_b2, 32, min(4, N)),
        (ob2_w1, ob2_b1, ob2_w2, ob2_b2, 16, min(16, N)),
        (ob3_w1, ob3_b1, ob3_w2, ob3_b2, 8, min(32, N)),
        (ob4_w1, ob4_b1, ob4_w2, ob4_b2, 4, min(64, N)),
        (ob5_w1, ob5_b1, ob5_w2, ob5_b2, 2, min(128, N)),
    ]
    cur = _downselect(allp, 64, 1, stages[0][5])
    for i, (w1, b1, w2, b2, S, B) in enumerate(stages):
        allp = _stage_call(cur, w3(w1), b1.reshape(1, -1),
                           w3(w2), b2.reshape(1, -1), _vmask(S), S, B)
        if i + 1 < len(stages):
            cur = _downselect(allp, S, B, stages[i + 1][5])
    Wp2 = _geo(2)[0]
    off = allp.reshape(N, 6, Wp2, 4)[:, 2:3, 1:2, :].astype(jnp.float32)

    pat = _patches7(x, 64)
    x_can = _to_can(x, 64, jnp.float32)
    out1_can = _pos_call(
        pat, whead(pos_head_w), pos_head_b.reshape(1, -1),
        body_w1.reshape(8, 3, 384, 128).astype(bf), body_b1.reshape(8, 1, 128),
        body_w2.reshape(8, 3, 384, 128).astype(bf), body_b2.reshape(8, 1, 128),
        w3(pos_tail_w), pos_tail_b.reshape(1, -1),
        off.reshape(N, 1, 4), x_can, mask64)
    Wp, _, _, _ = _geo(64)
    out1 = out1_can.reshape(N, 68, Wp, 1)[:, 2:66, 1:65, :]
    return out1, off
